# baseline (device time: 195331 ns/iter reference)
import jax
import jax.numpy as jnp
from jax import lax
from jax.experimental import pallas as pl
from jax.experimental.pallas import tpu as pltpu

N_DEV = 16
CW_HOPS = 8
CCW_HOPS = 7
ROW_BLK = 1024
N_FLOWS = 3


def kernel(x, w_mat, scale_x, scale_w):
    m, k_per = x.shape
    _, n = w_mat.shape
    hm = m // 2

    comm_dt = jnp.float8_e5m2
    x8 = x.astype(comm_dt)
    w8 = w_mat.astype(comm_dt)

    def body(x_ref, w_ref, sx_ref, sw_ref, out_ref,
             cwx, cww, ccwx, ccww,
             cw_ss, cw_rs, ccw_ss, ccw_rs):
        me = lax.axis_index("i")
        left = lax.rem(me + N_DEV - 1, N_DEV)
        right = lax.rem(me + 1, N_DEV)

        barrier_sem = pltpu.get_barrier_semaphore()
        for nbr in (left, right):
            pl.semaphore_signal(
                barrier_sem, inc=1,
                device_id=(nbr,), device_id_type=pl.DeviceIdType.MESH,
            )
        pl.semaphore_wait(barrier_sem, 2)

        def mk(cw, h, f):
            xbuf, wbuf = (cwx, cww) if cw else (ccwx, ccww)
            ss, rs = (cw_ss, cw_rs) if cw else (ccw_ss, ccw_rs)
            dev = right if cw else left
            if f < 2:
                src = (x_ref.at[pl.ds(f * hm, hm)] if h == 1
                       else xbuf.at[h - 1, pl.ds(f * hm, hm)])
                dst = xbuf.at[h, pl.ds(f * hm, hm)]
            else:
                src = w_ref if h == 1 else wbuf.at[h - 1]
                dst = wbuf.at[h]
            return pltpu.make_async_remote_copy(
                src_ref=src, dst_ref=dst,
                send_sem=ss.at[h - 1, f], recv_sem=rs.at[h - 1, f],
                device_id=(dev,), device_id_type=pl.DeviceIdType.MESH,
            )

        def acc_chunk(xr, wr, first=False):
            for r in range(0, m, ROW_BLK):
                part = jax.lax.dot_general(
                    xr[pl.ds(r, ROW_BLK), :], wr[...],
                    dimension_numbers=(((1,), (0,)), ((), ())),
                    preferred_element_type=jnp.float32,
                )
                if first:
                    out_ref[pl.ds(r, ROW_BLK), :] = part
                else:
                    out_ref[pl.ds(r, ROW_BLK), :] += part

        def hop_descs(h):
            ds = {}
            for f in range(N_FLOWS):
                ds[(True, f)] = mk(True, h, f)
                if h <= CCW_HOPS:
                    ds[(False, f)] = mk(False, h, f)
            return ds

        FLOW_ORDER = [(cw, f) for f in range(N_FLOWS) for cw in (True, False)]

        cur = hop_descs(1)
        for key in FLOW_ORDER:
            cur[key].start()
        acc_chunk(x_ref, w_ref, first=True)

        for h in range(1, CW_HOPS + 1):
            nxt = hop_descs(h + 1) if h < CW_HOPS else {}
            for key in FLOW_ORDER:
                if key not in cur:
                    continue
                cur[key].wait_recv()
                cur[key].wait_send()
                if key in nxt:
                    nxt[key].start()
            cur = nxt
            if h <= CCW_HOPS:
                wcat = jnp.concatenate([cww[h], ccww[h]], axis=0)
                for r in range(0, m, ROW_BLK):
                    xcat = jnp.concatenate(
                        [cwx[h, pl.ds(r, ROW_BLK), :],
                         ccwx[h, pl.ds(r, ROW_BLK), :]], axis=1)
                    out_ref[pl.ds(r, ROW_BLK), :] += jax.lax.dot_general(
                        xcat, wcat,
                        dimension_numbers=(((1,), (0,)), ((), ())),
                        preferred_element_type=jnp.float32,
                    )
            else:
                s = sx_ref[0] * sw_ref[0]
                for r in range(0, m, ROW_BLK):
                    acc = out_ref[pl.ds(r, ROW_BLK), :] + jax.lax.dot_general(
                        cwx[h, pl.ds(r, ROW_BLK), :], cww[h],
                        dimension_numbers=(((1,), (0,)), ((), ())),
                        preferred_element_type=jnp.float32,
                    )
                    y = acc * s
                    z = jnp.clip(y, -60.0, 60.0)
                    out_ref[pl.ds(r, ROW_BLK), :] = y / (1.0 + jnp.exp(-z))

    return pl.pallas_call(
        body,
        out_shape=jax.ShapeDtypeStruct((m, n), jnp.float32),
        in_specs=[
            pl.BlockSpec(memory_space=pltpu.VMEM),
            pl.BlockSpec(memory_space=pltpu.VMEM),
            pl.BlockSpec(memory_space=pltpu.SMEM),
            pl.BlockSpec(memory_space=pltpu.SMEM),
        ],
        out_specs=pl.BlockSpec(memory_space=pltpu.VMEM),
        scratch_shapes=[
            pltpu.VMEM((CW_HOPS + 1, m, k_per), comm_dt),
            pltpu.VMEM((CW_HOPS + 1, k_per, n), comm_dt),
            pltpu.VMEM((CCW_HOPS + 1, m, k_per), comm_dt),
            pltpu.VMEM((CCW_HOPS + 1, k_per, n), comm_dt),
            pltpu.SemaphoreType.DMA((CW_HOPS, N_FLOWS)),
            pltpu.SemaphoreType.DMA((CW_HOPS, N_FLOWS)),
            pltpu.SemaphoreType.DMA((CCW_HOPS, N_FLOWS)),
            pltpu.SemaphoreType.DMA((CCW_HOPS, N_FLOWS)),
        ],
        compiler_params=pltpu.CompilerParams(
            collective_id=0,
            vmem_limit_bytes=128 * 1024 * 1024,
        ),
    )(x8, w8, scale_x, scale_w)


# device time: 191870 ns/iter; 1.0180x vs baseline; 1.0180x over previous
import jax
import jax.numpy as jnp
from jax import lax
from jax.experimental import pallas as pl
from jax.experimental.pallas import tpu as pltpu

N_DEV = 16
CW_HOPS = 8
CCW_HOPS = 7
ROW_BLK = 1024
N_FLOWS = 3


def kernel(x, w_mat, scale_x, scale_w):
    m, k_per = x.shape
    _, n = w_mat.shape
    hm = m // 2

    comm_dt = jnp.float8_e5m2
    x8 = x.astype(comm_dt)
    w8 = w_mat.astype(comm_dt)

    def body(x_ref, w_ref, sx_ref, sw_ref, out_ref,
             cwx, cww, ccwx, ccww,
             cw_ss, cw_rs, ccw_ss, ccw_rs):
        me = lax.axis_index("i")
        left = lax.rem(me + N_DEV - 1, N_DEV)
        right = lax.rem(me + 1, N_DEV)

        barrier_sem = pltpu.get_barrier_semaphore()
        for nbr in (left, right):
            pl.semaphore_signal(
                barrier_sem, inc=1,
                device_id=(nbr,), device_id_type=pl.DeviceIdType.MESH,
            )
        pl.semaphore_wait(barrier_sem, 2)

        def mk(cw, h, f):
            xbuf, wbuf = (cwx, cww) if cw else (ccwx, ccww)
            ss, rs = (cw_ss, cw_rs) if cw else (ccw_ss, ccw_rs)
            dev = right if cw else left
            if f < 2:
                src = (x_ref.at[pl.ds(f * hm, hm)] if h == 1
                       else xbuf.at[h - 1, pl.ds(f * hm, hm)])
                dst = xbuf.at[h, pl.ds(f * hm, hm)]
            else:
                src = w_ref if h == 1 else wbuf.at[h - 1]
                dst = wbuf.at[h]
            return pltpu.make_async_remote_copy(
                src_ref=src, dst_ref=dst,
                send_sem=ss.at[h - 1, f], recv_sem=rs.at[h - 1, f],
                device_id=(dev,), device_id_type=pl.DeviceIdType.MESH,
            )

        def acc_chunk(xr, wr, first=False):
            for r in range(0, m, ROW_BLK):
                part = jax.lax.dot_general(
                    xr[pl.ds(r, ROW_BLK), :], wr[...],
                    dimension_numbers=(((1,), (0,)), ((), ())),
                    preferred_element_type=jnp.float32,
                )
                if first:
                    out_ref[pl.ds(r, ROW_BLK), :] = part
                else:
                    out_ref[pl.ds(r, ROW_BLK), :] += part

        def hop_descs(h):
            ds = {}
            for f in range(N_FLOWS):
                ds[(True, f)] = mk(True, h, f)
                if h <= CCW_HOPS:
                    ds[(False, f)] = mk(False, h, f)
            return ds

        FLOW_ORDER = [(cw, f) for f in (2, 0, 1) for cw in (True, False)]

        cur = hop_descs(1)
        for key in FLOW_ORDER:
            cur[key].start()
        acc_chunk(x_ref, w_ref, first=True)

        for h in range(1, CW_HOPS):
            nxt = hop_descs(h + 1)
            for key in FLOW_ORDER:
                if key not in cur:
                    continue
                cur[key].wait_recv()
                cur[key].wait_send()
                if key in nxt:
                    nxt[key].start()
            cur = nxt
            wcat = jnp.concatenate([cww[h], ccww[h]], axis=0)
            for r in range(0, m, ROW_BLK):
                xcat = jnp.concatenate(
                    [cwx[h, pl.ds(r, ROW_BLK), :],
                     ccwx[h, pl.ds(r, ROW_BLK), :]], axis=1)
                out_ref[pl.ds(r, ROW_BLK), :] += jax.lax.dot_general(
                    xcat, wcat,
                    dimension_numbers=(((1,), (0,)), ((), ())),
                    preferred_element_type=jnp.float32,
                )

        h = CW_HOPS
        s = sx_ref[0] * sw_ref[0]
        dw = cur[(True, 2)]
        dw.wait_recv()
        dw.wait_send()
        for f in (0, 1):
            d = cur[(True, f)]
            d.wait_recv()
            d.wait_send()
            for r in range(f * hm, (f + 1) * hm, ROW_BLK):
                acc = out_ref[pl.ds(r, ROW_BLK), :] + jax.lax.dot_general(
                    cwx[h, pl.ds(r, ROW_BLK), :], cww[h],
                    dimension_numbers=(((1,), (0,)), ((), ())),
                    preferred_element_type=jnp.float32,
                )
                y = acc * s
                z = jnp.clip(y, -60.0, 60.0)
                out_ref[pl.ds(r, ROW_BLK), :] = y / (1.0 + jnp.exp(-z))

    return pl.pallas_call(
        body,
        out_shape=jax.ShapeDtypeStruct((m, n), jnp.float32),
        in_specs=[
            pl.BlockSpec(memory_space=pltpu.VMEM),
            pl.BlockSpec(memory_space=pltpu.VMEM),
            pl.BlockSpec(memory_space=pltpu.SMEM),
            pl.BlockSpec(memory_space=pltpu.SMEM),
        ],
        out_specs=pl.BlockSpec(memory_space=pltpu.VMEM),
        scratch_shapes=[
            pltpu.VMEM((CW_HOPS + 1, m, k_per), comm_dt),
            pltpu.VMEM((CW_HOPS + 1, k_per, n), comm_dt),
            pltpu.VMEM((CCW_HOPS + 1, m, k_per), comm_dt),
            pltpu.VMEM((CCW_HOPS + 1, k_per, n), comm_dt),
            pltpu.SemaphoreType.DMA((CW_HOPS, N_FLOWS)),
            pltpu.SemaphoreType.DMA((CW_HOPS, N_FLOWS)),
            pltpu.SemaphoreType.DMA((CCW_HOPS, N_FLOWS)),
            pltpu.SemaphoreType.DMA((CCW_HOPS, N_FLOWS)),
        ],
        compiler_params=pltpu.CompilerParams(
            collective_id=0,
            vmem_limit_bytes=128 * 1024 * 1024,
        ),
    )(x8, w8, scale_x, scale_w)
